# baseline (speedup 1.0000x reference)
import functools

import jax
import jax.numpy as jnp
from jax import lax
from jax.experimental import pallas as pl
from jax.experimental.pallas import tpu as pltpu
from jax.experimental.pallas import tpu_sc as plsc

VOCAB = 65536
BATCH = 16384
DIM = 16

NUM_CORES = 2
NUM_SUBCORES = 16
LANES = 16
NUM_WORKERS = NUM_CORES * NUM_SUBCORES
B_PER_W = BATCH // NUM_WORKERS
CHUNK = 128
NCHUNK = B_PER_W // CHUNK
GROUPS = CHUNK // LANES

TC_BLK = 1024
NLINES = VOCAB // 8


def _tc_pack(src_t, dst_t):
    def body(s_ref, d_ref, so_ref, do_ref):
        eye = jnp.eye(128, dtype=jnp.float32)
        for x_ref, o_ref in ((s_ref, so_ref), (d_ref, do_ref)):
            z = jnp.reshape(x_ref[...], (128, 128))
            o_ref[...] = lax.dot_general(
                z, eye, (((0,), (0,)), ((), ())),
                preferred_element_type=jnp.float32)

    return pl.pallas_call(
        body,
        grid=(VOCAB // TC_BLK,),
        compiler_params=pltpu.CompilerParams(
            dimension_semantics=("parallel",)),
        in_specs=[
            pl.BlockSpec((DIM, TC_BLK), lambda k: (0, k)),
            pl.BlockSpec((DIM, TC_BLK), lambda k: (0, k)),
        ],
        out_specs=[
            pl.BlockSpec((128, 128), lambda k: (k, 0)),
            pl.BlockSpec((128, 128), lambda k: (k, 0)),
        ],
        out_shape=[
            jax.ShapeDtypeStruct((NLINES, 128), jnp.float32),
            jax.ShapeDtypeStruct((NLINES, 128), jnp.float32),
        ],
    )(src_t, dst_t)


def kernel(srcs, dsts, src_table, dst_table):
    src_lines, dst_lines = _tc_pack(
        jnp.transpose(src_table), jnp.transpose(dst_table))

    mesh = plsc.VectorSubcoreMesh(
        core_axis_name="c", subcore_axis_name="s",
        num_cores=NUM_CORES, num_subcores=NUM_SUBCORES)
    cp = pltpu.CompilerParams(
        needs_layout_passes=False, use_tc_tiling_on_sc=False)

    @functools.partial(
        pl.kernel,
        out_type=jax.ShapeDtypeStruct((BATCH,), jnp.float32),
        mesh=mesh,
        compiler_params=cp,
        scratch_types=[
            pltpu.VMEM((B_PER_W,), jnp.int32),
            pltpu.VMEM((B_PER_W,), jnp.int32),
            pltpu.VMEM((B_PER_W,), jnp.int32),
            pltpu.VMEM((B_PER_W,), jnp.int32),
            pltpu.VMEM((CHUNK, 128), jnp.float32),
            pltpu.VMEM((CHUNK, 128), jnp.float32),
            pltpu.VMEM((B_PER_W,), jnp.float32),
            pltpu.SemaphoreType.DMA,
            pltpu.SemaphoreType.DMA,
        ],
    )
    def sc_kernel(srcs_hbm, dsts_hbm, srct_hbm, dstt_hbm, out_hbm,
                  sidx_v, didx_v, sline_v, dline_v, sblk_v, dblk_v,
                  out_v, sem_s, sem_d):
        wid = lax.axis_index("s") * NUM_CORES + lax.axis_index("c")
        base = wid * B_PER_W
        cp_si = pltpu.async_copy(srcs_hbm.at[pl.ds(base, B_PER_W)], sidx_v, sem_s)
        cp_di = pltpu.async_copy(dsts_hbm.at[pl.ds(base, B_PER_W)], didx_v, sem_d)
        cp_si.wait()
        cp_di.wait()

        @pl.loop(0, B_PER_W // LANES)
        def _(g):
            sl = pl.ds(g * LANES, LANES)
            sg = sidx_v[sl]
            dg = didx_v[sl]
            sline_v[sl] = jax.lax.shift_left(
                jax.lax.shift_right_logical(sg, 10), 7) + jax.lax.bitwise_and(sg, 127)
            dline_v[sl] = jax.lax.shift_left(
                jax.lax.shift_right_logical(dg, 10), 7) + jax.lax.bitwise_and(dg, 127)

        lane = lax.iota(jnp.int32, LANES)

        @pl.loop(0, NCHUNK)
        def _(c):
            cbase = c * CHUNK
            cp_s = pltpu.async_copy(
                srct_hbm.at[sline_v.at[pl.ds(cbase, CHUNK)]], sblk_v, sem_s)
            cp_d = pltpu.async_copy(
                dstt_hbm.at[dline_v.at[pl.ds(cbase, CHUNK)]], dblk_v, sem_d)
            cp_s.wait()
            cp_d.wait()

            @pl.loop(0, GROUPS)
            def _(g):
                sl = pl.ds(cbase + g * LANES, LANES)
                sg = sidx_v[sl]
                dg = didx_v[sl]
                samp = lane + g * LANES
                se = jax.lax.bitwise_and(
                    jax.lax.shift_right_logical(sg, 7), 7)
                de = jax.lax.bitwise_and(
                    jax.lax.shift_right_logical(dg, 7), 7)
                acc = jnp.zeros((LANES,), jnp.float32)
                for d in range(DIM):
                    sv = plsc.load_gather(sblk_v, [samp, se + (8 * d)])
                    dv = plsc.load_gather(dblk_v, [samp, de + (8 * d)])
                    acc = acc + sv * dv
                out_v[sl] = acc

        pltpu.sync_copy(out_v, out_hbm.at[pl.ds(base, B_PER_W)])

    return sc_kernel(srcs, dsts, src_lines, dst_lines)


# speedup vs baseline: 1.8330x; 1.8330x over previous
import functools

import jax
import jax.numpy as jnp
from jax import lax
from jax.experimental import pallas as pl
from jax.experimental.pallas import tpu as pltpu
from jax.experimental.pallas import tpu_sc as plsc

VOCAB = 65536
BATCH = 16384
DIM = 16

NUM_CORES = 2
NUM_SUBCORES = 16
LANES = 16
NUM_WORKERS = NUM_CORES * NUM_SUBCORES
B_PER_W = BATCH // NUM_WORKERS
CHUNK = 128
NCHUNK = B_PER_W // CHUNK
GROUPS = CHUNK // LANES

TC_BLK = 8192
NLINES = VOCAB // 8


def _tc_pack(src_t, dst_t):
    sub = TC_BLK // 1024

    def body(s_ref, d_ref, so_ref, do_ref):
        for x_ref, o_ref in ((s_ref, so_ref), (d_ref, do_ref)):
            for t in range(sub):
                z = jnp.reshape(
                    x_ref[:, t * 1024:(t + 1) * 1024], (128, 128))
                o_ref[t * 128:(t + 1) * 128, :] = jnp.transpose(z)

    return pl.pallas_call(
        body,
        grid=(VOCAB // TC_BLK,),
        compiler_params=pltpu.CompilerParams(
            dimension_semantics=("parallel",)),
        in_specs=[
            pl.BlockSpec((DIM, TC_BLK), lambda k: (0, k)),
            pl.BlockSpec((DIM, TC_BLK), lambda k: (0, k)),
        ],
        out_specs=[
            pl.BlockSpec((TC_BLK // 8, 128), lambda k: (k, 0)),
            pl.BlockSpec((TC_BLK // 8, 128), lambda k: (k, 0)),
        ],
        out_shape=[
            jax.ShapeDtypeStruct((NLINES, 128), jnp.float32),
            jax.ShapeDtypeStruct((NLINES, 128), jnp.float32),
        ],
    )(src_t, dst_t)


def kernel(srcs, dsts, src_table, dst_table):
    src_lines, dst_lines = _tc_pack(
        jnp.transpose(src_table), jnp.transpose(dst_table))

    mesh = plsc.VectorSubcoreMesh(
        core_axis_name="c", subcore_axis_name="s",
        num_cores=NUM_CORES, num_subcores=NUM_SUBCORES)
    cp = pltpu.CompilerParams(
        needs_layout_passes=False, use_tc_tiling_on_sc=False)

    @functools.partial(
        pl.kernel,
        out_type=jax.ShapeDtypeStruct((BATCH,), jnp.float32),
        mesh=mesh,
        compiler_params=cp,
        scratch_types=[
            pltpu.VMEM((B_PER_W,), jnp.int32),
            pltpu.VMEM((B_PER_W,), jnp.int32),
            pltpu.VMEM((B_PER_W,), jnp.int32),
            pltpu.VMEM((B_PER_W,), jnp.int32),
            pltpu.VMEM((CHUNK, 128), jnp.float32),
            pltpu.VMEM((CHUNK, 128), jnp.float32),
            pltpu.VMEM((B_PER_W,), jnp.float32),
            pltpu.SemaphoreType.DMA,
            pltpu.SemaphoreType.DMA,
        ],
    )
    def sc_kernel(srcs_hbm, dsts_hbm, srct_hbm, dstt_hbm, out_hbm,
                  sidx_v, didx_v, sline_v, dline_v, sblk_v, dblk_v,
                  out_v, sem_s, sem_d):
        wid = lax.axis_index("s") * NUM_CORES + lax.axis_index("c")
        base = wid * B_PER_W
        cp_si = pltpu.async_copy(srcs_hbm.at[pl.ds(base, B_PER_W)], sidx_v, sem_s)
        cp_di = pltpu.async_copy(dsts_hbm.at[pl.ds(base, B_PER_W)], didx_v, sem_d)
        cp_si.wait()
        cp_di.wait()

        @pl.loop(0, B_PER_W // LANES)
        def _(g):
            sl = pl.ds(g * LANES, LANES)
            sg = sidx_v[sl]
            dg = didx_v[sl]
            sline_v[sl] = jax.lax.shift_left(
                jax.lax.shift_right_logical(sg, 10), 7) + jax.lax.bitwise_and(sg, 127)
            dline_v[sl] = jax.lax.shift_left(
                jax.lax.shift_right_logical(dg, 10), 7) + jax.lax.bitwise_and(dg, 127)

        lane = lax.iota(jnp.int32, LANES)

        @pl.loop(0, NCHUNK)
        def _(c):
            cbase = c * CHUNK
            cp_s = pltpu.async_copy(
                srct_hbm.at[sline_v.at[pl.ds(cbase, CHUNK)]], sblk_v, sem_s)
            cp_d = pltpu.async_copy(
                dstt_hbm.at[dline_v.at[pl.ds(cbase, CHUNK)]], dblk_v, sem_d)
            cp_s.wait()
            cp_d.wait()

            @pl.loop(0, GROUPS)
            def _(g):
                sl = pl.ds(cbase + g * LANES, LANES)
                sg = sidx_v[sl]
                dg = didx_v[sl]
                samp = lane + g * LANES
                se = jax.lax.bitwise_and(
                    jax.lax.shift_right_logical(sg, 7), 7)
                de = jax.lax.bitwise_and(
                    jax.lax.shift_right_logical(dg, 7), 7)
                acc = jnp.zeros((LANES,), jnp.float32)
                for d in range(DIM):
                    sv = plsc.load_gather(sblk_v, [samp, se + (8 * d)])
                    dv = plsc.load_gather(dblk_v, [samp, de + (8 * d)])
                    acc = acc + sv * dv
                out_v[sl] = acc

        pltpu.sync_copy(out_v, out_hbm.at[pl.ds(base, B_PER_W)])

    return sc_kernel(srcs, dsts, src_lines, dst_lines)


# speedup vs baseline: 1.9494x; 1.0635x over previous
import functools

import jax
import jax.numpy as jnp
from jax import lax
from jax.experimental import pallas as pl
from jax.experimental.pallas import tpu as pltpu
from jax.experimental.pallas import tpu_sc as plsc

VOCAB = 65536
BATCH = 16384
DIM = 16

NUM_CORES = 2
NUM_SUBCORES = 16
LANES = 16
NUM_WORKERS = NUM_CORES * NUM_SUBCORES
B_PER_W = BATCH // NUM_WORKERS
CHUNK = 128
NCHUNK = B_PER_W // CHUNK
GROUPS = CHUNK // LANES

TC_BLK = 16384
NLINES = VOCAB // 8


def _tc_pack(src_t, dst_t):
    sub = TC_BLK // 1024

    def body(s_ref, d_ref, so_ref, do_ref):
        for x_ref, o_ref in ((s_ref, so_ref), (d_ref, do_ref)):
            for t in range(sub):
                z = jnp.reshape(
                    x_ref[:, t * 1024:(t + 1) * 1024], (128, 128))
                o_ref[t * 128:(t + 1) * 128, :] = jnp.transpose(z)

    return pl.pallas_call(
        body,
        grid=(VOCAB // TC_BLK,),
        compiler_params=pltpu.CompilerParams(
            dimension_semantics=("parallel",)),
        in_specs=[
            pl.BlockSpec((DIM, TC_BLK), lambda k: (0, k)),
            pl.BlockSpec((DIM, TC_BLK), lambda k: (0, k)),
        ],
        out_specs=[
            pl.BlockSpec((TC_BLK // 8, 128), lambda k: (k, 0)),
            pl.BlockSpec((TC_BLK // 8, 128), lambda k: (k, 0)),
        ],
        out_shape=[
            jax.ShapeDtypeStruct((NLINES, 128), jnp.float32),
            jax.ShapeDtypeStruct((NLINES, 128), jnp.float32),
        ],
    )(src_t, dst_t)


def kernel(srcs, dsts, src_table, dst_table):
    src_lines, dst_lines = _tc_pack(
        jnp.transpose(src_table), jnp.transpose(dst_table))

    mesh = plsc.VectorSubcoreMesh(
        core_axis_name="c", subcore_axis_name="s",
        num_cores=NUM_CORES, num_subcores=NUM_SUBCORES)
    cp = pltpu.CompilerParams(
        needs_layout_passes=False, use_tc_tiling_on_sc=False)

    @functools.partial(
        pl.kernel,
        out_type=jax.ShapeDtypeStruct((BATCH,), jnp.float32),
        mesh=mesh,
        compiler_params=cp,
        scratch_types=[
            pltpu.VMEM((B_PER_W,), jnp.int32),
            pltpu.VMEM((B_PER_W,), jnp.int32),
            pltpu.VMEM((B_PER_W,), jnp.int32),
            pltpu.VMEM((B_PER_W,), jnp.int32),
            pltpu.VMEM((CHUNK, 128), jnp.float32),
            pltpu.VMEM((CHUNK, 128), jnp.float32),
            pltpu.VMEM((CHUNK, 128), jnp.float32),
            pltpu.VMEM((CHUNK, 128), jnp.float32),
            pltpu.VMEM((B_PER_W,), jnp.float32),
            pltpu.SemaphoreType.DMA,
            pltpu.SemaphoreType.DMA,
            pltpu.SemaphoreType.DMA,
            pltpu.SemaphoreType.DMA,
        ],
    )
    def sc_kernel(srcs_hbm, dsts_hbm, srct_hbm, dstt_hbm, out_hbm,
                  sidx_v, didx_v, sline_v, dline_v,
                  sblk0_v, sblk1_v, dblk0_v, dblk1_v,
                  out_v, sem_s0, sem_s1, sem_d0, sem_d1):
        wid = lax.axis_index("s") * NUM_CORES + lax.axis_index("c")
        base = wid * B_PER_W
        cp_si = pltpu.async_copy(srcs_hbm.at[pl.ds(base, B_PER_W)], sidx_v, sem_s0)
        cp_di = pltpu.async_copy(dsts_hbm.at[pl.ds(base, B_PER_W)], didx_v, sem_d0)
        cp_si.wait()
        cp_di.wait()

        @pl.loop(0, B_PER_W // LANES)
        def _(g):
            sl = pl.ds(g * LANES, LANES)
            sg = sidx_v[sl]
            dg = didx_v[sl]
            sline_v[sl] = jax.lax.shift_left(
                jax.lax.shift_right_logical(sg, 10), 7) + jax.lax.bitwise_and(sg, 127)
            dline_v[sl] = jax.lax.shift_left(
                jax.lax.shift_right_logical(dg, 10), 7) + jax.lax.bitwise_and(dg, 127)

        lane = lax.iota(jnp.int32, LANES)
        sbufs = (sblk0_v, sblk1_v)
        dbufs = (dblk0_v, dblk1_v)
        ssems = (sem_s0, sem_s1)
        dsems = (sem_d0, sem_d1)

        def issue(c):
            b = c % 2
            sl = pl.ds(c * CHUNK, CHUNK)
            return (
                pltpu.async_copy(srct_hbm.at[sline_v.at[sl]], sbufs[b], ssems[b]),
                pltpu.async_copy(dstt_hbm.at[dline_v.at[sl]], dbufs[b], dsems[b]),
            )

        cps = {0: issue(0)}
        for c in range(NCHUNK):
            if c + 1 < NCHUNK:
                cps[c + 1] = issue(c + 1)
            cp_s, cp_d = cps.pop(c)
            cp_s.wait()
            cp_d.wait()
            sblk_v = sbufs[c % 2]
            dblk_v = dbufs[c % 2]
            for g in range(GROUPS):
                sl = pl.ds(c * CHUNK + g * LANES, LANES)
                sg = sidx_v[sl]
                dg = didx_v[sl]
                samp = lane + g * LANES
                se = jax.lax.bitwise_and(
                    jax.lax.shift_right_logical(sg, 7), 7)
                de = jax.lax.bitwise_and(
                    jax.lax.shift_right_logical(dg, 7), 7)
                acc = jnp.zeros((LANES,), jnp.float32)
                for d in range(DIM):
                    sv = plsc.load_gather(sblk_v, [samp, se + (8 * d)])
                    dv = plsc.load_gather(dblk_v, [samp, de + (8 * d)])
                    acc = acc + sv * dv
                out_v[sl] = acc

        pltpu.sync_copy(out_v, out_hbm.at[pl.ds(base, B_PER_W)])

    return sc_kernel(srcs, dsts, src_lines, dst_lines)


# speedup vs baseline: 2.0301x; 1.0414x over previous
import functools

import jax
import jax.numpy as jnp
from jax import lax
from jax.experimental import pallas as pl
from jax.experimental.pallas import tpu as pltpu
from jax.experimental.pallas import tpu_sc as plsc

VOCAB = 65536
BATCH = 16384
DIM = 16

NUM_CORES = 2
NUM_SUBCORES = 16
LANES = 16
NUM_WORKERS = NUM_CORES * NUM_SUBCORES
B_PER_W = BATCH // NUM_WORKERS
CHUNK = 128
NCHUNK = B_PER_W // CHUNK
GROUPS = CHUNK // LANES

TC_BLK = 32768
NLINES = VOCAB // 8


def _tc_pack(src_t, dst_t):
    sub = TC_BLK // 1024

    def body(s_ref, d_ref, so_ref, do_ref):
        for x_ref, o_ref in ((s_ref, so_ref), (d_ref, do_ref)):
            for t in range(sub):
                z = jnp.reshape(
                    x_ref[:, t * 1024:(t + 1) * 1024], (128, 128))
                o_ref[t * 128:(t + 1) * 128, :] = jnp.transpose(z)

    return pl.pallas_call(
        body,
        grid=(VOCAB // TC_BLK,),
        compiler_params=pltpu.CompilerParams(
            dimension_semantics=("parallel",)),
        in_specs=[
            pl.BlockSpec((DIM, TC_BLK), lambda k: (0, k)),
            pl.BlockSpec((DIM, TC_BLK), lambda k: (0, k)),
        ],
        out_specs=[
            pl.BlockSpec((TC_BLK // 8, 128), lambda k: (k, 0)),
            pl.BlockSpec((TC_BLK // 8, 128), lambda k: (k, 0)),
        ],
        out_shape=[
            jax.ShapeDtypeStruct((NLINES, 128), jnp.float32),
            jax.ShapeDtypeStruct((NLINES, 128), jnp.float32),
        ],
    )(src_t, dst_t)


def kernel(srcs, dsts, src_table, dst_table):
    src_lines, dst_lines = _tc_pack(
        jnp.transpose(src_table), jnp.transpose(dst_table))

    mesh = plsc.VectorSubcoreMesh(
        core_axis_name="c", subcore_axis_name="s",
        num_cores=NUM_CORES, num_subcores=NUM_SUBCORES)
    cp = pltpu.CompilerParams(
        needs_layout_passes=False, use_tc_tiling_on_sc=False)

    @functools.partial(
        pl.kernel,
        out_type=jax.ShapeDtypeStruct((BATCH,), jnp.float32),
        mesh=mesh,
        compiler_params=cp,
        scratch_types=[
            pltpu.VMEM((B_PER_W,), jnp.int32),
            pltpu.VMEM((B_PER_W,), jnp.int32),
            pltpu.VMEM((B_PER_W,), jnp.int32),
            pltpu.VMEM((B_PER_W,), jnp.int32),
            pltpu.VMEM((CHUNK, 128), jnp.float32),
            pltpu.VMEM((CHUNK, 128), jnp.float32),
            pltpu.VMEM((CHUNK, 128), jnp.float32),
            pltpu.VMEM((CHUNK, 128), jnp.float32),
            pltpu.VMEM((B_PER_W,), jnp.float32),
            pltpu.SemaphoreType.DMA,
            pltpu.SemaphoreType.DMA,
            pltpu.SemaphoreType.DMA,
            pltpu.SemaphoreType.DMA,
        ],
    )
    def sc_kernel(srcs_hbm, dsts_hbm, srct_hbm, dstt_hbm, out_hbm,
                  sidx_v, didx_v, sline_v, dline_v,
                  sblk0_v, sblk1_v, dblk0_v, dblk1_v,
                  out_v, sem_s0, sem_s1, sem_d0, sem_d1):
        wid = lax.axis_index("s") * NUM_CORES + lax.axis_index("c")
        base = wid * B_PER_W
        cp_si = pltpu.async_copy(srcs_hbm.at[pl.ds(base, B_PER_W)], sidx_v, sem_s0)
        cp_di = pltpu.async_copy(dsts_hbm.at[pl.ds(base, B_PER_W)], didx_v, sem_d0)
        cp_si.wait()
        cp_di.wait()

        @pl.loop(0, B_PER_W // LANES)
        def _(g):
            sl = pl.ds(g * LANES, LANES)
            sg = sidx_v[sl]
            dg = didx_v[sl]
            sline_v[sl] = jax.lax.shift_left(
                jax.lax.shift_right_logical(sg, 10), 7) + jax.lax.bitwise_and(sg, 127)
            dline_v[sl] = jax.lax.shift_left(
                jax.lax.shift_right_logical(dg, 10), 7) + jax.lax.bitwise_and(dg, 127)

        lane = lax.iota(jnp.int32, LANES)
        sbufs = (sblk0_v, sblk1_v)
        dbufs = (dblk0_v, dblk1_v)
        ssems = (sem_s0, sem_s1)
        dsems = (sem_d0, sem_d1)

        def issue(c):
            b = c % 2
            sl = pl.ds(c * CHUNK, CHUNK)
            return (
                pltpu.async_copy(srct_hbm.at[sline_v.at[sl]], sbufs[b], ssems[b]),
                pltpu.async_copy(dstt_hbm.at[dline_v.at[sl]], dbufs[b], dsems[b]),
            )

        cps = {0: issue(0)}
        for c in range(NCHUNK):
            if c + 1 < NCHUNK:
                cps[c + 1] = issue(c + 1)
            cp_s, cp_d = cps.pop(c)
            cp_s.wait()
            cp_d.wait()
            sblk_v = sbufs[c % 2]
            dblk_v = dbufs[c % 2]
            for g in range(GROUPS):
                sl = pl.ds(c * CHUNK + g * LANES, LANES)
                sg = sidx_v[sl]
                dg = didx_v[sl]
                samp = lane + g * LANES
                se = jax.lax.bitwise_and(
                    jax.lax.shift_right_logical(sg, 7), 7)
                de = jax.lax.bitwise_and(
                    jax.lax.shift_right_logical(dg, 7), 7)
                acc = jnp.zeros((LANES,), jnp.float32)
                for d in range(DIM):
                    sv = plsc.load_gather(sblk_v, [samp, se + (8 * d)])
                    dv = plsc.load_gather(dblk_v, [samp, de + (8 * d)])
                    acc = acc + sv * dv
                out_v[sl] = acc

        pltpu.sync_copy(out_v, out_hbm.at[pl.ds(base, B_PER_W)])

    return sc_kernel(srcs, dsts, src_lines, dst_lines)
